# baseline (speedup 1.0000x reference)
import functools

import jax
import jax.numpy as jnp
from jax.experimental import pallas as pl
from jax.experimental.pallas import tpu as pltpu

TEMP = 0.1
EPS = 1e-8
KPAD = 24


def _body(x0_ref, x1_ref, x2_ref, x3_ref, lab_ref, rel_ref, core_ref, tr_ref,
          out_ref, pn_ref, acc_ref, *, K, P, T, C):
    b = pl.program_id(0)
    s = pl.program_id(1)

    @pl.when(jnp.logical_and(b == 0, s == 0))
    def _init():
        cc = jnp.concatenate([core_ref[...], tr_ref[...]], axis=1)
        cc = cc / (jnp.sqrt(jnp.sum(cc * cc, axis=2, keepdims=True)) + EPS)
        pn_ref[:, :K, :] = jnp.transpose(cc, (1, 0, 2)).astype(jnp.bfloat16)
        acc_ref[0] = 0.0
        acc_ref[1] = 0.0

    x = jnp.concatenate(
        [r[0].astype(jnp.bfloat16).reshape(C // 4, T)
         for r in (x0_ref, x1_ref, x2_ref, x3_ref)], axis=0)
    xsq = x * x
    colsq = jnp.dot(jnp.ones((1, C), jnp.bfloat16), xsq,
                    preferred_element_type=jnp.float32)
    inv = 1.0 / ((jnp.sqrt(colsq) + EPS) * TEMP)
    mm = jnp.dot(pn_ref[...].reshape(P * KPAD, C), x,
                 preferred_element_type=jnp.float32)
    cl = jnp.max(mm.reshape(P, KPAD, T), axis=0) * inv
    kidx = jax.lax.broadcasted_iota(jnp.int32, (KPAD, T), 0)
    cl = jnp.where(kidx < K, cl, -1e4)
    lse = jnp.log(jnp.sum(jnp.exp(cl), axis=0, keepdims=True))
    lab = lab_ref[0].reshape(1, T)
    label_logit = jnp.sum(jnp.where(kidx == lab, cl, 0.0), axis=0, keepdims=True)
    nll = lse - label_logit
    w = rel_ref[0, 0].reshape(1, T)
    acc_ref[0] += jnp.sum(nll * w)
    acc_ref[1] += jnp.sum(w)

    @pl.when(jnp.logical_and(b == pl.num_programs(0) - 1,
                             s == pl.num_programs(1) - 1))
    def _fin():
        out_ref[0, 0] = acc_ref[0] / (acc_ref[1] + EPS)


def kernel(proj, labels, core_prototypes, transition_prototypes, reliability_map):
    B, C, H, W = proj.shape
    S = H * W
    K, Pc, _ = core_prototypes.shape
    P = Pc + transition_prototypes.shape[1]

    Hb = 64
    T = Hb * W
    grid = (B, S // T)

    out = pl.pallas_call(
        functools.partial(_body, K=K, P=P, T=T, C=C),
        grid=grid,
        in_specs=[
            pl.BlockSpec((1, C // 4, Hb, W), lambda b, s: (b, 0, s, 0)),
            pl.BlockSpec((1, C // 4, Hb, W), lambda b, s: (b, 1, s, 0)),
            pl.BlockSpec((1, C // 4, Hb, W), lambda b, s: (b, 2, s, 0)),
            pl.BlockSpec((1, C // 4, Hb, W), lambda b, s: (b, 3, s, 0)),
            pl.BlockSpec((1, Hb, W), lambda b, s: (b, s, 0)),
            pl.BlockSpec((1, 1, Hb, W), lambda b, s: (b, 0, s, 0)),
            pl.BlockSpec((K, Pc, C), lambda b, s: (0, 0, 0)),
            pl.BlockSpec((K, P - Pc, C), lambda b, s: (0, 0, 0)),
        ],
        out_specs=pl.BlockSpec((1, 1), lambda b, s: (0, 0),
                               memory_space=pltpu.SMEM),
        out_shape=jax.ShapeDtypeStruct((1, 1), jnp.float32),
        scratch_shapes=[
            pltpu.VMEM((P, KPAD, C), jnp.bfloat16),
            pltpu.SMEM((2,), jnp.float32),
        ],
    )(proj, proj, proj, proj, labels, reliability_map,
      core_prototypes, transition_prototypes)
    return out.reshape(())


# speedup vs baseline: 1.0031x; 1.0031x over previous
import functools

import jax
import jax.numpy as jnp
from jax.experimental import pallas as pl
from jax.experimental.pallas import tpu as pltpu

TEMP = 0.1
EPS = 1e-8
KPAD = 24


def _body(x_ref, lab_ref, rel_ref, core_ref, tr_ref, out_ref, pn_ref, acc_ref,
          *, K, P, T, C):
    b = pl.program_id(0)
    s = pl.program_id(1)

    @pl.when(jnp.logical_and(b == 0, s == 0))
    def _init():
        cc = jnp.concatenate([core_ref[...], tr_ref[...]], axis=1)
        cc = cc / (jnp.sqrt(jnp.sum(cc * cc, axis=2, keepdims=True)) + EPS)
        pn_ref[:, :K, :] = jnp.transpose(cc, (1, 0, 2)).astype(jnp.bfloat16)
        acc_ref[0] = 0.0
        acc_ref[1] = 0.0

    x = x_ref[0].astype(jnp.bfloat16).reshape(C, T)
    xsq = x * x
    colsq = jnp.dot(jnp.ones((1, C), jnp.bfloat16), xsq,
                    preferred_element_type=jnp.float32)
    inv = 1.0 / ((jnp.sqrt(colsq) + EPS) * TEMP)
    mm = jnp.dot(pn_ref[...].reshape(P * KPAD, C), x,
                 preferred_element_type=jnp.float32)
    cl = jnp.max(mm.reshape(P, KPAD, T), axis=0) * inv
    kidx = jax.lax.broadcasted_iota(jnp.int32, (KPAD, T), 0)
    cl = jnp.where(kidx < K, cl, -1e4)
    lse = jnp.log(jnp.sum(jnp.exp(cl), axis=0, keepdims=True))
    lab = lab_ref[0].reshape(1, T)
    label_logit = jnp.sum(jnp.where(kidx == lab, cl, 0.0), axis=0, keepdims=True)
    nll = lse - label_logit
    w = rel_ref[0, 0].reshape(1, T)
    acc_ref[0] += jnp.sum(nll * w)
    acc_ref[1] += jnp.sum(w)

    @pl.when(jnp.logical_and(b == pl.num_programs(0) - 1,
                             s == pl.num_programs(1) - 1))
    def _fin():
        out_ref[0, 0] = acc_ref[0] / (acc_ref[1] + EPS)


def kernel(proj, labels, core_prototypes, transition_prototypes, reliability_map):
    B, C, H, W = proj.shape
    S = H * W
    K, Pc, _ = core_prototypes.shape
    P = Pc + transition_prototypes.shape[1]

    Hb = 64
    T = Hb * W
    grid = (B, S // T)

    out = pl.pallas_call(
        functools.partial(_body, K=K, P=P, T=T, C=C),
        grid=grid,
        in_specs=[
            pl.BlockSpec((1, C, Hb, W), lambda b, s: (b, 0, s, 0)),
            pl.BlockSpec((1, Hb, W), lambda b, s: (b, s, 0)),
            pl.BlockSpec((1, 1, Hb, W), lambda b, s: (b, 0, s, 0)),
            pl.BlockSpec((K, Pc, C), lambda b, s: (0, 0, 0)),
            pl.BlockSpec((K, P - Pc, C), lambda b, s: (0, 0, 0)),
        ],
        out_specs=pl.BlockSpec((1, 1), lambda b, s: (0, 0),
                               memory_space=pltpu.SMEM),
        out_shape=jax.ShapeDtypeStruct((1, 1), jnp.float32),
        scratch_shapes=[
            pltpu.VMEM((P, KPAD, C), jnp.bfloat16),
            pltpu.SMEM((2,), jnp.float32),
        ],
    )(proj, labels, reliability_map, core_prototypes, transition_prototypes)
    return out.reshape(())
